# baseline (speedup 1.0000x reference)
import functools

import jax
import jax.numpy as jnp
from jax import lax
from jax.experimental import pallas as pl
from jax.experimental.pallas import tpu as pltpu
from jax.experimental.pallas import tpu_sc as plsc

T = 2048
D = 1024
E = 8
H = 1536
K = 2
A = T * K
M = 256
MSH = 8
NP = A + E * M
NT = NP // M
LANES = 16
NW = 32
CPW = A // NW
TPW = T // NW

_EPAD = 128


def _router_body(x_ref, rw_ref, ew_ref, ei_ref):
    logits = jnp.dot(x_ref[...], rw_ref[...], preferred_element_type=jnp.float32)
    lane = lax.broadcasted_iota(jnp.int32, logits.shape, 1)
    valid = lane < E
    logits = jnp.where(valid, logits, -1e30)
    m = jnp.max(logits, axis=1, keepdims=True)
    ex = jnp.where(valid, jnp.exp(logits - m), 0.0)
    sm = ex / jnp.sum(ex, axis=1, keepdims=True)
    v1 = jnp.max(sm, axis=1, keepdims=True)
    i1 = jnp.min(jnp.where(sm == v1, lane, E), axis=1, keepdims=True)
    smm = jnp.where(lane == i1, -1.0, sm)
    v2 = jnp.max(smm, axis=1, keepdims=True)
    i2 = jnp.min(jnp.where(smm == v2, lane, E), axis=1, keepdims=True)
    ew_ref[...] = jnp.where(lane == 0, v1, jnp.where(lane == 1, v2, 0.0))
    ei_ref[...] = jnp.where(lane == 0, i1, jnp.where(lane == 1, i2, 0))


def _router(t, rwp):
    return pl.pallas_call(
        _router_body,
        out_shape=(
            jax.ShapeDtypeStruct((T, _EPAD), jnp.float32),
            jax.ShapeDtypeStruct((T, _EPAD), jnp.int32),
        ),
    )(t, rwp)


def _dispatch_body(ei_hbm, x_hbm, xd_hbm, inv_hbm, tmeta_hbm,
                   ei_v, inv_v, rowbuf, tmeta_v, sem):
    cid = lax.axis_index("c")
    sid = lax.axis_index("s")
    wid = sid * 2 + cid
    pltpu.sync_copy(ei_hbm, ei_v)
    lanes = lax.iota(jnp.int32, LANES)
    zero = jnp.zeros((LANES,), jnp.int32)
    my_chunk0 = wid * (CPW // LANES)

    def count_body(c, carry):
        cnt, pref = carry
        pref = jnp.where(jnp.broadcast_to(c == my_chunk0, (LANES,)), cnt, pref)
        v = ei_v[pl.ds(c * LANES, LANES)]
        for e in range(E):
            pc = plsc.all_reduce_population_count(v == e)
            cnt = cnt + jnp.where(lanes == e, pc, 0)
        return cnt, pref

    tot, pref = lax.fori_loop(0, A // LANES, count_body, (zero, zero))

    padded = ((tot + (M - 1)) >> MSH) << MSH
    incl = plsc.cumsum(padded)
    base = (incl - padded) + pref

    for ci in range(CPW // LANES):
        off = wid * CPW + ci * LANES
        v = ei_v[pl.ds(off, LANES)]
        pos = zero
        for e in range(E):
            msk = v == e
            ones = jnp.where(msk, 1, 0)
            csum = plsc.cumsum(ones)
            be = jnp.sum(jnp.where(lanes == e, base, 0))
            pos = jnp.where(msk, be + csum - 1, pos)
            base = base + jnp.where(lanes == e, jnp.sum(ones), 0)
        tok = (lanes + off) >> 1
        pltpu.async_copy(x_hbm.at[tok], rowbuf, sem).wait()
        pltpu.async_copy(rowbuf, xd_hbm.at[pos], sem).wait()
        inv_v[pl.ds(ci * LANES, LANES)] = pos
    pltpu.sync_copy(inv_v, inv_hbm.at[pl.ds(wid * CPW, CPW)])

    @pl.when(jnp.logical_and(cid == 0, sid == 0))
    def _():
        j0 = lanes * M
        j1 = (lanes + LANES) * M
        te0 = zero
        te1 = zero
        for e in range(E):
            se = jnp.sum(jnp.where(lanes == e, incl, 0))
            te0 = te0 + jnp.where(j0 >= se, 1, 0)
            te1 = te1 + jnp.where(j1 >= se, 1, 0)
        te0 = jnp.minimum(te0, E - 1)
        te1 = jnp.minimum(te1, E - 1)
        used = jnp.sum(jnp.where(lanes == E - 1, incl, 0)) >> MSH
        te1 = jnp.where(lanes + LANES < NT, te1, used)
        tmeta_v[pl.ds(0, LANES)] = te0
        tmeta_v[pl.ds(LANES, LANES)] = te1
        pltpu.sync_copy(tmeta_v, tmeta_hbm)


@functools.cache
def _sc_kernels():
    mesh = plsc.VectorSubcoreMesh(core_axis_name="c", subcore_axis_name="s")
    dispatch = pl.kernel(
        _dispatch_body,
        out_type=(
            jax.ShapeDtypeStruct((NP, D), jnp.float32),
            jax.ShapeDtypeStruct((A,), jnp.int32),
            jax.ShapeDtypeStruct((2 * LANES,), jnp.int32),
        ),
        mesh=mesh,
        compiler_params=pltpu.CompilerParams(needs_layout_passes=False),
        scratch_types=[
            pltpu.VMEM((A,), jnp.int32),
            pltpu.VMEM((CPW,), jnp.int32),
            pltpu.VMEM((LANES, D), jnp.float32),
            pltpu.VMEM((2 * LANES,), jnp.int32),
            pltpu.SemaphoreType.DMA,
        ],
    )
    combine = pl.kernel(
        _combine_body,
        out_type=jax.ShapeDtypeStruct((T, D), jnp.float32),
        mesh=mesh,
        scratch_types=[
            pltpu.VMEM((CPW,), jnp.int32),
            pltpu.VMEM((CPW + LANES,), jnp.float32),
            pltpu.VMEM((D,), jnp.float32),
            pltpu.VMEM((2 * LANES, D), jnp.float32),
            pltpu.VMEM((LANES, D), jnp.float32),
            pltpu.SemaphoreType.DMA,
        ],
    )
    return dispatch, combine


def _gemm_body(tmeta_ref, xd_ref, w1_ref, v1_ref, w2_ref, out_ref):
    i = pl.program_id(0)
    used = tmeta_ref[24]

    @pl.when(i < used)
    def _():
        xb = xd_ref[...].astype(jnp.bfloat16)
        g = jnp.dot(xb, w1_ref[0].astype(jnp.bfloat16),
                    preferred_element_type=jnp.float32)
        u = jnp.dot(xb, v1_ref[0].astype(jnp.bfloat16),
                    preferred_element_type=jnp.float32)
        h = (0.5 * g * (1.0 + lax.erf(g * 0.7071067811865476))) * u
        out_ref[...] = jnp.dot(h.astype(jnp.bfloat16), w2_ref[0].astype(jnp.bfloat16),
                               preferred_element_type=jnp.float32)


def _gemm(tmeta, xd, w1, v1, w2):
    grid_spec = pltpu.PrefetchScalarGridSpec(
        num_scalar_prefetch=1,
        grid=(NT,),
        in_specs=[
            pl.BlockSpec((M, D), lambda i, tm: (i, 0)),
            pl.BlockSpec((1, D, H), lambda i, tm: (tm[i], 0, 0)),
            pl.BlockSpec((1, D, H), lambda i, tm: (tm[i], 0, 0)),
            pl.BlockSpec((1, H, D), lambda i, tm: (tm[i], 0, 0)),
        ],
        out_specs=pl.BlockSpec((M, D), lambda i, tm: (i, 0)),
    )
    return pl.pallas_call(
        _gemm_body,
        grid_spec=grid_spec,
        out_shape=jax.ShapeDtypeStruct((NP, D), jnp.float32),
    )(tmeta, xd, w1, v1, w2)


def _combine_body(yd_hbm, inv_hbm, ew_hbm, b_hbm, out_hbm,
                  inv_v, ew_v, bias_v, ybuf, obuf, sem):
    cid = lax.axis_index("c")
    sid = lax.axis_index("s")
    wid = sid * 2 + cid
    pltpu.sync_copy(inv_hbm.at[pl.ds(wid * CPW, CPW)], inv_v)
    pltpu.sync_copy(ew_hbm.at[pl.ds(wid * CPW, CPW)], ew_v.at[pl.ds(0, CPW)])
    pltpu.sync_copy(b_hbm, bias_v)
    for b in range(TPW // LANES):
        idx0 = inv_v[pl.ds(b * 32, LANES)]
        idx1 = inv_v[pl.ds(b * 32 + LANES, LANES)]
        pltpu.async_copy(yd_hbm.at[idx0], ybuf.at[pl.ds(0, LANES)], sem).wait()
        pltpu.async_copy(yd_hbm.at[idx1], ybuf.at[pl.ds(LANES, LANES)], sem).wait()

        for i in range(LANES):
            wv = ew_v[pl.ds(b * 32 + 2 * i, LANES)]
            w0 = wv[0]
            w1_ = wv[1]

            def cc_body(cc, inner, i=i, w0=w0, w1_=w1_):
                y0 = ybuf[2 * i, pl.ds(cc * LANES, LANES)]
                y1 = ybuf[2 * i + 1, pl.ds(cc * LANES, LANES)]
                bb = bias_v[pl.ds(cc * LANES, LANES)]
                obuf[i, pl.ds(cc * LANES, LANES)] = y0 * w0 + y1 * w1_ + bb
                return inner

            lax.fori_loop(0, D // LANES, cc_body, jnp.int32(0))
        pltpu.sync_copy(obuf, out_hbm.at[pl.ds(wid * TPW + b * LANES, LANES)])


def kernel(x, router_w, w1, v1, w2, bias):
    t = x.reshape(T, D)
    rwp = jnp.zeros((D, _EPAD), jnp.float32).at[:, :E].set(router_w)
    ew_pad, ei_pad = _router(t, rwp)
    ew = ew_pad[:, :K].reshape(A)
    ei = ei_pad[:, :K].reshape(A)
    dispatch, combine = _sc_kernels()
    xd, inv, tmeta = dispatch(ei, t)
    yd = _gemm(tmeta, xd, w1, v1, w2)
    out = combine(yd, inv, ew, bias)
    return out.reshape(1, T, D)


# speedup vs baseline: 1.0029x; 1.0029x over previous
import functools

import jax
import jax.numpy as jnp
from jax import lax
from jax.experimental import pallas as pl
from jax.experimental.pallas import tpu as pltpu
from jax.experimental.pallas import tpu_sc as plsc

T = 2048
D = 1024
E = 8
H = 1536
K = 2
A = T * K
M = 256
MSH = 8
NP = A + E * M
NT = NP // M
LANES = 16
NW = 32
CPW = A // NW
TPW = T // NW

_EPAD = 128


def _router_body(x_ref, rw_ref, ew_ref, ei_ref):
    logits = jnp.dot(x_ref[...], rw_ref[...], preferred_element_type=jnp.float32)
    lane = lax.broadcasted_iota(jnp.int32, logits.shape, 1)
    valid = lane < E
    logits = jnp.where(valid, logits, -1e30)
    m = jnp.max(logits, axis=1, keepdims=True)
    ex = jnp.where(valid, jnp.exp(logits - m), 0.0)
    sm = ex / jnp.sum(ex, axis=1, keepdims=True)
    v1 = jnp.max(sm, axis=1, keepdims=True)
    i1 = jnp.min(jnp.where(sm == v1, lane, E), axis=1, keepdims=True)
    smm = jnp.where(lane == i1, -1.0, sm)
    v2 = jnp.max(smm, axis=1, keepdims=True)
    i2 = jnp.min(jnp.where(smm == v2, lane, E), axis=1, keepdims=True)
    ew_ref[...] = jnp.where(lane == 0, v1, jnp.where(lane == 1, v2, 0.0))
    ei_ref[...] = jnp.where(lane == 0, i1, jnp.where(lane == 1, i2, 0))


def _router(t, rwp):
    return pl.pallas_call(
        _router_body,
        out_shape=(
            jax.ShapeDtypeStruct((T, _EPAD), jnp.float32),
            jax.ShapeDtypeStruct((T, _EPAD), jnp.int32),
        ),
    )(t, rwp)


_NCH = CPW // LANES
_NSLOT = 6


def _dispatch_body(ei_hbm, x_hbm, xd_hbm, inv_hbm, tmeta_hbm,
                   ei_v, inv_v, rowbuf, tmeta_v, semg, sems):
    cid = lax.axis_index("c")
    sid = lax.axis_index("s")
    wid = sid * 2 + cid
    pltpu.sync_copy(ei_hbm, ei_v)
    lanes = lax.iota(jnp.int32, LANES)
    zero = jnp.zeros((LANES,), jnp.int32)
    my_chunk0 = wid * _NCH

    gh = []
    for ci in range(_NSLOT):
        tok = (lanes + wid * CPW + ci * LANES) >> 1
        gh.append(pltpu.async_copy(
            x_hbm.at[tok], rowbuf.at[pl.ds(ci * LANES, LANES)], semg))

    def count_body(c, carry):
        cnt, pref = carry
        pref = jnp.where(jnp.broadcast_to(c == my_chunk0, (LANES,)), cnt, pref)
        v = ei_v[pl.ds(c * LANES, LANES)]
        for e in range(E):
            pc = plsc.all_reduce_population_count(v == e)
            cnt = cnt + jnp.where(lanes == e, pc, 0)
        return cnt, pref

    tot, pref = lax.fori_loop(0, A // LANES, count_body, (zero, zero))

    padded = ((tot + (M - 1)) >> MSH) << MSH
    incl = plsc.cumsum(padded)
    base = (incl - padded) + pref

    sh = [None] * _NCH
    for ci in range(_NCH):
        off = wid * CPW + ci * LANES
        v = ei_v[pl.ds(off, LANES)]
        pos = zero
        for e in range(E):
            msk = v == e
            ones = jnp.where(msk, 1, 0)
            csum = plsc.cumsum(ones)
            be = jnp.sum(jnp.where(lanes == e, base, 0))
            pos = jnp.where(msk, be + csum - 1, pos)
            base = base + jnp.where(lanes == e, jnp.sum(ones), 0)
        inv_v[pl.ds(ci * LANES, LANES)] = pos
        gh[ci].wait()
        slot = ci % _NSLOT
        sh[ci] = pltpu.async_copy(
            rowbuf.at[pl.ds(slot * LANES, LANES)], xd_hbm.at[pos], sems)
        nxt = ci - 1 + _NSLOT
        if ci >= 1 and nxt < _NCH and len(gh) == nxt:
            sh[ci - 1].wait()
            sh[ci - 1] = None
            tok = (lanes + wid * CPW + nxt * LANES) >> 1
            gh.append(pltpu.async_copy(
                x_hbm.at[tok],
                rowbuf.at[pl.ds(((ci - 1) % _NSLOT) * LANES, LANES)], semg))
    for h in sh:
        if h is not None:
            h.wait()
    pltpu.sync_copy(inv_v, inv_hbm.at[pl.ds(wid * CPW, CPW)])

    @pl.when(jnp.logical_and(cid == 0, sid == 0))
    def _():
        j0 = lanes * M
        j1 = (lanes + LANES) * M
        te0 = zero
        te1 = zero
        for e in range(E):
            se = jnp.sum(jnp.where(lanes == e, incl, 0))
            te0 = te0 + jnp.where(j0 >= se, 1, 0)
            te1 = te1 + jnp.where(j1 >= se, 1, 0)
        te0 = jnp.minimum(te0, E - 1)
        te1 = jnp.minimum(te1, E - 1)
        used = jnp.sum(jnp.where(lanes == E - 1, incl, 0)) >> MSH
        te1 = jnp.where(lanes + LANES < NT, te1, used)
        tmeta_v[pl.ds(0, LANES)] = te0
        tmeta_v[pl.ds(LANES, LANES)] = te1
        pltpu.sync_copy(tmeta_v, tmeta_hbm)


@functools.cache
def _sc_kernels():
    mesh = plsc.VectorSubcoreMesh(core_axis_name="c", subcore_axis_name="s")
    dispatch = pl.kernel(
        _dispatch_body,
        out_type=(
            jax.ShapeDtypeStruct((NP, D), jnp.float32),
            jax.ShapeDtypeStruct((A,), jnp.int32),
            jax.ShapeDtypeStruct((2 * LANES,), jnp.int32),
        ),
        mesh=mesh,
        compiler_params=pltpu.CompilerParams(needs_layout_passes=False),
        scratch_types=[
            pltpu.VMEM((A,), jnp.int32),
            pltpu.VMEM((CPW,), jnp.int32),
            pltpu.VMEM((_NSLOT * LANES, D), jnp.float32),
            pltpu.VMEM((2 * LANES,), jnp.int32),
            pltpu.SemaphoreType.DMA,
            pltpu.SemaphoreType.DMA,
        ],
    )
    combine = pl.kernel(
        _combine_body,
        out_type=jax.ShapeDtypeStruct((T, D), jnp.float32),
        mesh=mesh,
        scratch_types=[
            pltpu.VMEM((CPW,), jnp.int32),
            pltpu.VMEM((CPW + LANES,), jnp.float32),
            pltpu.VMEM((D,), jnp.float32),
            pltpu.VMEM((4 * LANES, D), jnp.float32),
            pltpu.VMEM((2 * LANES, D), jnp.float32),
            pltpu.SemaphoreType.DMA,
            pltpu.SemaphoreType.DMA,
        ],
    )
    return dispatch, combine


def _gemm_body(tmeta_ref, xd_ref, w1_ref, v1_ref, w2_ref, out_ref):
    i = pl.program_id(0)
    used = tmeta_ref[24]

    @pl.when(i < used)
    def _():
        xb = xd_ref[...].astype(jnp.bfloat16)
        g = jnp.dot(xb, w1_ref[0].astype(jnp.bfloat16),
                    preferred_element_type=jnp.float32)
        u = jnp.dot(xb, v1_ref[0].astype(jnp.bfloat16),
                    preferred_element_type=jnp.float32)
        h = (0.5 * g * (1.0 + lax.erf(g * 0.7071067811865476))) * u
        out_ref[...] = jnp.dot(h.astype(jnp.bfloat16), w2_ref[0].astype(jnp.bfloat16),
                               preferred_element_type=jnp.float32)


def _gemm(tmeta, xd, w1, v1, w2):
    grid_spec = pltpu.PrefetchScalarGridSpec(
        num_scalar_prefetch=1,
        grid=(NT,),
        in_specs=[
            pl.BlockSpec((M, D), lambda i, tm: (i, 0)),
            pl.BlockSpec((1, D, H), lambda i, tm: (tm[i], 0, 0)),
            pl.BlockSpec((1, D, H), lambda i, tm: (tm[i], 0, 0)),
            pl.BlockSpec((1, H, D), lambda i, tm: (tm[i], 0, 0)),
        ],
        out_specs=pl.BlockSpec((M, D), lambda i, tm: (i, 0)),
    )
    return pl.pallas_call(
        _gemm_body,
        grid_spec=grid_spec,
        out_shape=jax.ShapeDtypeStruct((NP, D), jnp.float32),
    )(tmeta, xd, w1, v1, w2)


def _combine_body(yd_hbm, inv_hbm, ew_hbm, b_hbm, out_hbm,
                  inv_v, ew_v, bias_v, ybuf, obuf, semg, semo):
    cid = lax.axis_index("c")
    sid = lax.axis_index("s")
    wid = sid * 2 + cid
    NB = TPW // LANES
    pltpu.sync_copy(inv_hbm.at[pl.ds(wid * CPW, CPW)], inv_v)
    pltpu.sync_copy(ew_hbm.at[pl.ds(wid * CPW, CPW)], ew_v.at[pl.ds(0, CPW)])
    pltpu.sync_copy(b_hbm, bias_v)

    def fire(b):
        idx0 = inv_v[pl.ds(b * 32, LANES)]
        idx1 = inv_v[pl.ds(b * 32 + LANES, LANES)]
        s = (b % 2) * 32
        h0 = pltpu.async_copy(yd_hbm.at[idx0],
                              ybuf.at[pl.ds(s, LANES)], semg)
        h1 = pltpu.async_copy(yd_hbm.at[idx1],
                              ybuf.at[pl.ds(s + LANES, LANES)], semg)
        return h0, h1

    hs = {0: fire(0)}
    oh = [None] * NB
    for b in range(NB):
        if b + 1 < NB:
            hs[b + 1] = fire(b + 1)
        hs[b][0].wait()
        hs[b][1].wait()
        if b >= 2:
            oh[b - 2].wait()
        yrow = (b % 2) * 32
        orow = (b % 2) * LANES
        for i in range(LANES):
            wv = ew_v[pl.ds(b * 32 + 2 * i, LANES)]
            w0 = wv[0]
            w1_ = wv[1]

            def cc_body(cc, inner, i=i, w0=w0, w1_=w1_, yrow=yrow, orow=orow):
                for u in range(4):
                    c0 = cc * (4 * LANES) + u * LANES
                    y0 = ybuf[yrow + 2 * i, pl.ds(c0, LANES)]
                    y1 = ybuf[yrow + 2 * i + 1, pl.ds(c0, LANES)]
                    bb = bias_v[pl.ds(c0, LANES)]
                    obuf[orow + i, pl.ds(c0, LANES)] = y0 * w0 + y1 * w1_ + bb
                return inner

            lax.fori_loop(0, D // (4 * LANES), cc_body, jnp.int32(0))
        oh[b] = pltpu.async_copy(
            obuf.at[pl.ds(orow, LANES)],
            out_hbm.at[pl.ds(wid * TPW + b * LANES, LANES)], semo)
    oh[NB - 2].wait()
    oh[NB - 1].wait()


def kernel(x, router_w, w1, v1, w2, bias):
    t = x.reshape(T, D)
    rwp = jnp.zeros((D, _EPAD), jnp.float32).at[:, :E].set(router_w)
    ew_pad, ei_pad = _router(t, rwp)
    ew = ew_pad[:, :K].reshape(A)
    ei = ei_pad[:, :K].reshape(A)
    dispatch, combine = _sc_kernels()
    xd, inv, tmeta = dispatch(ei, t)
    yd = _gemm(tmeta, xd, w1, v1, w2)
    out = combine(yd, inv, ew, bias)
    return out.reshape(1, T, D)


# speedup vs baseline: 1.1878x; 1.1844x over previous
import functools

import jax
import jax.numpy as jnp
from jax import lax
from jax.experimental import pallas as pl
from jax.experimental.pallas import tpu as pltpu
from jax.experimental.pallas import tpu_sc as plsc

T = 2048
D = 1024
E = 8
H = 1536
K = 2
A = T * K
M = 256
MSH = 8
NP = A + E * M
NT = NP // M
LANES = 16
NW = 32
CPW = A // NW
TPW = T // NW

_EPAD = 128
D2 = D // 2


def _pack_bf16_pair(lo, hi):
    lo_b = lax.bitcast_convert_type(
        lo.astype(jnp.bfloat16).astype(jnp.float32), jnp.int32)
    hi_b = lax.bitcast_convert_type(
        hi.astype(jnp.bfloat16).astype(jnp.float32), jnp.int32)
    return jnp.bitwise_and(jnp.right_shift(lo_b, 16), 0xFFFF) | hi_b


def _router_body(x_ref, rw_ref, ew_ref, ei_ref, xb_ref):
    xw = x_ref[...]
    xb_ref[...] = _pack_bf16_pair(xw[:, :D2], xw[:, D2:])
    logits = jnp.dot(xw, rw_ref[...], preferred_element_type=jnp.float32)
    lane = lax.broadcasted_iota(jnp.int32, logits.shape, 1)
    valid = lane < E
    logits = jnp.where(valid, logits, -1e30)
    m = jnp.max(logits, axis=1, keepdims=True)
    ex = jnp.where(valid, jnp.exp(logits - m), 0.0)
    sm = ex / jnp.sum(ex, axis=1, keepdims=True)
    v1 = jnp.max(sm, axis=1, keepdims=True)
    i1 = jnp.min(jnp.where(sm == v1, lane, E), axis=1, keepdims=True)
    smm = jnp.where(lane == i1, -1.0, sm)
    v2 = jnp.max(smm, axis=1, keepdims=True)
    i2 = jnp.min(jnp.where(smm == v2, lane, E), axis=1, keepdims=True)
    ew_ref[...] = jnp.where(lane == 0, v1, jnp.where(lane == 1, v2, 0.0))
    ei_ref[...] = jnp.where(lane == 0, i1, jnp.where(lane == 1, i2, 0))


def _router(t, rwp):
    return pl.pallas_call(
        _router_body,
        out_shape=(
            jax.ShapeDtypeStruct((T, _EPAD), jnp.float32),
            jax.ShapeDtypeStruct((T, _EPAD), jnp.int32),
            jax.ShapeDtypeStruct((T, D2), jnp.int32),
        ),
    )(t, rwp)


_NCH = CPW // LANES
_NSLOT = 6


def _dispatch_body(ei_hbm, x_hbm, xd_hbm, inv_hbm, tmeta_hbm,
                   ei_v, inv_v, rowbuf, tmeta_v, semg, sems):
    cid = lax.axis_index("c")
    sid = lax.axis_index("s")
    wid = sid * 2 + cid
    pltpu.sync_copy(ei_hbm, ei_v)
    lanes = lax.iota(jnp.int32, LANES)
    zero = jnp.zeros((LANES,), jnp.int32)
    my_chunk0 = wid * _NCH

    gh = []
    for ci in range(_NSLOT):
        tok = (lanes + wid * CPW + ci * LANES) >> 1
        gh.append(pltpu.async_copy(
            x_hbm.at[tok], rowbuf.at[pl.ds(ci * LANES, LANES)], semg))

    def count_body(c, carry):
        cnt, pref = carry
        pref = jnp.where(jnp.broadcast_to(c == my_chunk0, (LANES,)), cnt, pref)
        v = ei_v[pl.ds(c * LANES, LANES)]
        for e in range(E):
            pc = plsc.all_reduce_population_count(v == e)
            cnt = cnt + jnp.where(lanes == e, pc, 0)
        return cnt, pref

    tot, pref = lax.fori_loop(0, A // LANES, count_body, (zero, zero))

    padded = ((tot + (M - 1)) >> MSH) << MSH
    incl = plsc.cumsum(padded)
    base = (incl - padded) + pref

    sh = [None] * _NCH
    for ci in range(_NCH):
        off = wid * CPW + ci * LANES
        v = ei_v[pl.ds(off, LANES)]
        pos = zero
        for e in range(E):
            msk = v == e
            ones = jnp.where(msk, 1, 0)
            csum = plsc.cumsum(ones)
            be = jnp.sum(jnp.where(lanes == e, base, 0))
            pos = jnp.where(msk, be + csum - 1, pos)
            base = base + jnp.where(lanes == e, jnp.sum(ones), 0)
        inv_v[pl.ds(ci * LANES, LANES)] = pos
        gh[ci].wait()
        slot = ci % _NSLOT
        sh[ci] = pltpu.async_copy(
            rowbuf.at[pl.ds(slot * LANES, LANES)], xd_hbm.at[pos], sems)
        nxt = ci - 1 + _NSLOT
        if ci >= 1 and nxt < _NCH and len(gh) == nxt:
            sh[ci - 1].wait()
            sh[ci - 1] = None
            tok = (lanes + wid * CPW + nxt * LANES) >> 1
            gh.append(pltpu.async_copy(
                x_hbm.at[tok],
                rowbuf.at[pl.ds(((ci - 1) % _NSLOT) * LANES, LANES)], semg))
    for h in sh:
        if h is not None:
            h.wait()
    pltpu.sync_copy(inv_v, inv_hbm.at[pl.ds(wid * CPW, CPW)])

    @pl.when(jnp.logical_and(cid == 0, sid == 0))
    def _():
        j0 = lanes * M
        j1 = (lanes + LANES) * M
        te0 = zero
        te1 = zero
        for e in range(E):
            se = jnp.sum(jnp.where(lanes == e, incl, 0))
            te0 = te0 + jnp.where(j0 >= se, 1, 0)
            te1 = te1 + jnp.where(j1 >= se, 1, 0)
        te0 = jnp.minimum(te0, E - 1)
        te1 = jnp.minimum(te1, E - 1)
        used = jnp.sum(jnp.where(lanes == E - 1, incl, 0)) >> MSH
        te1 = jnp.where(lanes + LANES < NT, te1, used)
        tmeta_v[pl.ds(0, LANES)] = te0
        tmeta_v[pl.ds(LANES, LANES)] = te1
        pltpu.sync_copy(tmeta_v, tmeta_hbm)


@functools.cache
def _sc_kernels():
    mesh = plsc.VectorSubcoreMesh(core_axis_name="c", subcore_axis_name="s")
    dispatch = pl.kernel(
        _dispatch_body,
        out_type=(
            jax.ShapeDtypeStruct((NP, D2), jnp.int32),
            jax.ShapeDtypeStruct((A,), jnp.int32),
            jax.ShapeDtypeStruct((2 * LANES,), jnp.int32),
        ),
        mesh=mesh,
        compiler_params=pltpu.CompilerParams(needs_layout_passes=False),
        scratch_types=[
            pltpu.VMEM((A,), jnp.int32),
            pltpu.VMEM((CPW,), jnp.int32),
            pltpu.VMEM((_NSLOT * LANES, D2), jnp.int32),
            pltpu.VMEM((2 * LANES,), jnp.int32),
            pltpu.SemaphoreType.DMA,
            pltpu.SemaphoreType.DMA,
        ],
    )
    combine = pl.kernel(
        _combine_body,
        out_type=jax.ShapeDtypeStruct((T, D), jnp.float32),
        mesh=mesh,
        compiler_params=pltpu.CompilerParams(needs_layout_passes=False),
        scratch_types=[
            pltpu.VMEM((CPW,), jnp.int32),
            pltpu.VMEM((CPW + LANES,), jnp.float32),
            pltpu.VMEM((D,), jnp.float32),
            pltpu.VMEM((4 * LANES, D2), jnp.int32),
            pltpu.VMEM((2 * LANES, D), jnp.float32),
            pltpu.SemaphoreType.DMA,
            pltpu.SemaphoreType.DMA,
        ],
    )
    return dispatch, combine


def _gemm_body(tmeta_ref, xd_ref, w1_ref, v1_ref, w2_ref, out_ref):
    i = pl.program_id(0)
    used = tmeta_ref[24]

    @pl.when(i < used)
    def _():
        xi = xd_ref[...]
        xa = lax.bitcast_convert_type(
            jnp.left_shift(xi, 16), jnp.float32).astype(jnp.bfloat16)
        xb = lax.bitcast_convert_type(
            jnp.bitwise_and(xi, -65536), jnp.float32).astype(jnp.bfloat16)
        w1b = w1_ref[0].astype(jnp.bfloat16)
        v1b = v1_ref[0].astype(jnp.bfloat16)
        g = (jnp.dot(xa, w1b[:D2], preferred_element_type=jnp.float32)
             + jnp.dot(xb, w1b[D2:], preferred_element_type=jnp.float32))
        u = (jnp.dot(xa, v1b[:D2], preferred_element_type=jnp.float32)
             + jnp.dot(xb, v1b[D2:], preferred_element_type=jnp.float32))
        h = (0.5 * g * (1.0 + lax.erf(g * 0.7071067811865476))) * u
        y = jnp.dot(h.astype(jnp.bfloat16), w2_ref[0].astype(jnp.bfloat16),
                    preferred_element_type=jnp.float32)
        out_ref[...] = _pack_bf16_pair(y[:, :D2], y[:, D2:])


def _gemm(tmeta, xd, w1, v1, w2):
    grid_spec = pltpu.PrefetchScalarGridSpec(
        num_scalar_prefetch=1,
        grid=(NT,),
        in_specs=[
            pl.BlockSpec((M, D2), lambda i, tm: (i, 0)),
            pl.BlockSpec((1, D, H), lambda i, tm: (tm[i], 0, 0)),
            pl.BlockSpec((1, D, H), lambda i, tm: (tm[i], 0, 0)),
            pl.BlockSpec((1, H, D), lambda i, tm: (tm[i], 0, 0)),
        ],
        out_specs=pl.BlockSpec((M, D2), lambda i, tm: (i, 0)),
    )
    return pl.pallas_call(
        _gemm_body,
        grid_spec=grid_spec,
        out_shape=jax.ShapeDtypeStruct((NP, D2), jnp.int32),
    )(tmeta, xd, w1, v1, w2)


def _combine_body(yd_hbm, inv_hbm, ew_hbm, b_hbm, out_hbm,
                  inv_v, ew_v, bias_v, ybuf, obuf, semg, semo):
    cid = lax.axis_index("c")
    sid = lax.axis_index("s")
    wid = sid * 2 + cid
    NB = TPW // LANES
    pltpu.sync_copy(inv_hbm.at[pl.ds(wid * CPW, CPW)], inv_v)
    pltpu.sync_copy(ew_hbm.at[pl.ds(wid * CPW, CPW)], ew_v.at[pl.ds(0, CPW)])
    pltpu.sync_copy(b_hbm, bias_v)

    def fire(b):
        idx0 = inv_v[pl.ds(b * 32, LANES)]
        idx1 = inv_v[pl.ds(b * 32 + LANES, LANES)]
        s = (b % 2) * 32
        h0 = pltpu.async_copy(yd_hbm.at[idx0],
                              ybuf.at[pl.ds(s, LANES)], semg)
        h1 = pltpu.async_copy(yd_hbm.at[idx1],
                              ybuf.at[pl.ds(s + LANES, LANES)], semg)
        return h0, h1

    hs = {0: fire(0)}
    oh = [None] * NB
    for b in range(NB):
        if b + 1 < NB:
            hs[b + 1] = fire(b + 1)
        hs[b][0].wait()
        hs[b][1].wait()
        if b >= 2:
            oh[b - 2].wait()
        yrow = (b % 2) * 32
        orow = (b % 2) * LANES

        def tok_body(i, carry, b=b, yrow=yrow, orow=orow):
            wv = ew_v[pl.ds(b * 32 + 2 * i, LANES)]
            w0 = wv[0]
            w1_ = wv[1]
            r0 = yrow + 2 * i
            ro = orow + i
            for cc in range(D2 // LANES):
                c0 = cc * LANES
                p0 = ybuf[r0, pl.ds(c0, LANES)]
                p1 = ybuf[r0 + 1, pl.ds(c0, LANES)]
                y0a = plsc.bitcast(jnp.left_shift(p0, 16), jnp.float32)
                y1a = plsc.bitcast(jnp.left_shift(p1, 16), jnp.float32)
                y0b = plsc.bitcast(jnp.bitwise_and(p0, -65536), jnp.float32)
                y1b = plsc.bitcast(jnp.bitwise_and(p1, -65536), jnp.float32)
                ba = bias_v[pl.ds(c0, LANES)]
                bb = bias_v[pl.ds(D2 + c0, LANES)]
                obuf[ro, pl.ds(c0, LANES)] = y0a * w0 + y1a * w1_ + ba
                obuf[ro, pl.ds(D2 + c0, LANES)] = y0b * w0 + y1b * w1_ + bb
            return carry

        lax.fori_loop(0, LANES, tok_body, jnp.int32(0))
        oh[b] = pltpu.async_copy(
            obuf.at[pl.ds(orow, LANES)],
            out_hbm.at[pl.ds(wid * TPW + b * LANES, LANES)], semo)
    oh[NB - 2].wait()
    oh[NB - 1].wait()


def kernel(x, router_w, w1, v1, w2, bias):
    t = x.reshape(T, D)
    rwp = jnp.zeros((D, _EPAD), jnp.float32).at[:, :E].set(router_w)
    ew_pad, ei_pad, tb = _router(t, rwp)
    ew = ew_pad[:, :K].reshape(A)
    ei = ei_pad[:, :K].reshape(A)
    dispatch, combine = _sc_kernels()
    xd, inv, tmeta = dispatch(ei, tb)
    yd = _gemm(tmeta, xd, w1, v1, w2)
    out = combine(yd, inv, ew, bias)
    return out.reshape(1, T, D)


# speedup vs baseline: 1.2044x; 1.0139x over previous
import functools

import jax
import jax.numpy as jnp
from jax import lax
from jax.experimental import pallas as pl
from jax.experimental.pallas import tpu as pltpu
from jax.experimental.pallas import tpu_sc as plsc

T = 2048
D = 1024
E = 8
H = 1536
K = 2
A = T * K
M = 256
MSH = 8
NP = A + E * M
NT = NP // M
LANES = 16
NW = 32
CPW = A // NW
TPW = T // NW

_EPAD = 128
D2 = D // 2


def _lane_bcast(v, idx):
    dn = lax.GatherDimensionNumbers(
        offset_dims=(), collapsed_slice_dims=(0,), start_index_map=(0,))
    return lax.gather(v, idx[:, None], dn, slice_sizes=(1,),
                      mode=lax.GatherScatterMode.PROMISE_IN_BOUNDS)


def _pack_bf16_pair(lo, hi):
    lo_b = lax.bitcast_convert_type(
        lo.astype(jnp.bfloat16).astype(jnp.float32), jnp.int32)
    hi_b = lax.bitcast_convert_type(
        hi.astype(jnp.bfloat16).astype(jnp.float32), jnp.int32)
    return jnp.bitwise_and(jnp.right_shift(lo_b, 16), 0xFFFF) | hi_b


def _router_body(x_ref, rw_ref, ew_ref, ei_ref, xb_ref):
    xw = x_ref[...]
    xb_ref[...] = _pack_bf16_pair(xw[:, :D2], xw[:, D2:])
    logits = jnp.dot(xw, rw_ref[...], preferred_element_type=jnp.float32)
    lane = lax.broadcasted_iota(jnp.int32, logits.shape, 1)
    valid = lane < E
    logits = jnp.where(valid, logits, -1e30)
    m = jnp.max(logits, axis=1, keepdims=True)
    ex = jnp.where(valid, jnp.exp(logits - m), 0.0)
    sm = ex / jnp.sum(ex, axis=1, keepdims=True)
    v1 = jnp.max(sm, axis=1, keepdims=True)
    i1 = jnp.min(jnp.where(sm == v1, lane, E), axis=1, keepdims=True)
    smm = jnp.where(lane == i1, -1.0, sm)
    v2 = jnp.max(smm, axis=1, keepdims=True)
    i2 = jnp.min(jnp.where(smm == v2, lane, E), axis=1, keepdims=True)
    ew_ref[...] = jnp.concatenate([v1, v2], axis=1)
    ei_ref[...] = jnp.concatenate([i1, i2], axis=1)


def _router(t, rwp):
    return pl.pallas_call(
        _router_body,
        out_shape=(
            jax.ShapeDtypeStruct((T, K), jnp.float32),
            jax.ShapeDtypeStruct((T, K), jnp.int32),
            jax.ShapeDtypeStruct((T, D2), jnp.int32),
        ),
    )(t, rwp)


_NCH = CPW // LANES
_NSLOT = 6


def _dispatch_body(ei_hbm, x_hbm, xd_hbm, inv_hbm, tmeta_hbm,
                   ei_v, inv_v, rowbuf, tmeta_v, semg, sems):
    cid = lax.axis_index("c")
    sid = lax.axis_index("s")
    wid = sid * 2 + cid
    pltpu.sync_copy(ei_hbm, ei_v)
    lanes = lax.iota(jnp.int32, LANES)
    zero = jnp.zeros((LANES,), jnp.int32)
    my_chunk0 = wid * _NCH

    gh = []
    for ci in range(_NSLOT):
        tok = (lanes + wid * CPW + ci * LANES) >> 1
        gh.append(pltpu.async_copy(
            x_hbm.at[tok], rowbuf.at[pl.ds(ci * LANES, LANES)], semg))

    def count_body(c, carry):
        cnt, pref = carry
        pref = jnp.where(jnp.broadcast_to(c == my_chunk0, (LANES,)), cnt, pref)
        v = ei_v[pl.ds(c * LANES, LANES)]
        for e in range(E):
            pc = plsc.all_reduce_population_count(v == e)
            cnt = cnt + jnp.where(lanes == e, pc, 0)
        return cnt, pref

    tot, pref = lax.fori_loop(0, A // LANES, count_body, (zero, zero))

    padded = ((tot + (M - 1)) >> MSH) << MSH
    incl = plsc.cumsum(padded)
    base = (incl - padded) + pref

    sh = [None] * _NCH
    for ci in range(_NCH):
        off = wid * CPW + ci * LANES
        v = ei_v[pl.ds(off, LANES)]
        pos = zero
        for e in range(E):
            msk = v == e
            ones = jnp.where(msk, 1, 0)
            csum = plsc.cumsum(ones)
            be = jnp.sum(jnp.where(lanes == e, base, 0))
            pos = jnp.where(msk, be + csum - 1, pos)
            base = base + jnp.where(lanes == e, jnp.sum(ones), 0)
        inv_v[pl.ds(ci * LANES, LANES)] = pos
        gh[ci].wait()
        slot = ci % _NSLOT
        sh[ci] = pltpu.async_copy(
            rowbuf.at[pl.ds(slot * LANES, LANES)], xd_hbm.at[pos], sems)
        nxt = ci - 1 + _NSLOT
        if ci >= 1 and nxt < _NCH and len(gh) == nxt:
            sh[ci - 1].wait()
            sh[ci - 1] = None
            tok = (lanes + wid * CPW + nxt * LANES) >> 1
            gh.append(pltpu.async_copy(
                x_hbm.at[tok],
                rowbuf.at[pl.ds(((ci - 1) % _NSLOT) * LANES, LANES)], semg))
    for h in sh:
        if h is not None:
            h.wait()
    pltpu.sync_copy(inv_v, inv_hbm.at[pl.ds(wid * CPW, CPW)])

    @pl.when(jnp.logical_and(cid == 0, sid == 0))
    def _():
        j0 = lanes * M
        j1 = (lanes + LANES) * M
        te0 = zero
        te1 = zero
        for e in range(E):
            se = jnp.sum(jnp.where(lanes == e, incl, 0))
            te0 = te0 + jnp.where(j0 >= se, 1, 0)
            te1 = te1 + jnp.where(j1 >= se, 1, 0)
        te0 = jnp.minimum(te0, E - 1)
        te1 = jnp.minimum(te1, E - 1)
        used = jnp.sum(jnp.where(lanes == E - 1, incl, 0)) >> MSH
        te1 = jnp.where(lanes + LANES < NT, te1, used)
        tmeta_v[pl.ds(0, LANES)] = te0
        tmeta_v[pl.ds(LANES, LANES)] = te1
        pltpu.sync_copy(tmeta_v, tmeta_hbm)


@functools.cache
def _sc_kernels():
    mesh = plsc.VectorSubcoreMesh(core_axis_name="c", subcore_axis_name="s")
    dispatch = pl.kernel(
        _dispatch_body,
        out_type=(
            jax.ShapeDtypeStruct((NP, D2), jnp.int32),
            jax.ShapeDtypeStruct((A,), jnp.int32),
            jax.ShapeDtypeStruct((2 * LANES,), jnp.int32),
        ),
        mesh=mesh,
        compiler_params=pltpu.CompilerParams(needs_layout_passes=False),
        scratch_types=[
            pltpu.VMEM((A,), jnp.int32),
            pltpu.VMEM((CPW,), jnp.int32),
            pltpu.VMEM((_NSLOT * LANES, D2), jnp.int32),
            pltpu.VMEM((2 * LANES,), jnp.int32),
            pltpu.SemaphoreType.DMA,
            pltpu.SemaphoreType.DMA,
        ],
    )
    combine = pl.kernel(
        _combine_body,
        out_type=jax.ShapeDtypeStruct((T, D), jnp.float32),
        mesh=mesh,
        compiler_params=pltpu.CompilerParams(needs_layout_passes=False),
        scratch_types=[
            pltpu.VMEM((CPW,), jnp.int32),
            pltpu.VMEM((CPW + LANES,), jnp.float32),
            pltpu.VMEM((D,), jnp.float32),
            pltpu.VMEM((4 * LANES, D2), jnp.int32),
            pltpu.VMEM((2 * LANES, D), jnp.float32),
            pltpu.SemaphoreType.DMA,
            pltpu.SemaphoreType.DMA,
        ],
    )
    return dispatch, combine


def _gemm_body(tmeta_ref, xd_ref, w1_ref, v1_ref, w2_ref, out_ref):
    i = pl.program_id(0)
    used = tmeta_ref[24]

    @pl.when(i < used)
    def _():
        xi = xd_ref[...]
        xa = lax.bitcast_convert_type(
            jnp.left_shift(xi, 16), jnp.float32).astype(jnp.bfloat16)
        xb = lax.bitcast_convert_type(
            jnp.bitwise_and(xi, -65536), jnp.float32).astype(jnp.bfloat16)
        w1b = w1_ref[0].astype(jnp.bfloat16)
        v1b = v1_ref[0].astype(jnp.bfloat16)
        g = (jnp.dot(xa, w1b[:D2], preferred_element_type=jnp.float32)
             + jnp.dot(xb, w1b[D2:], preferred_element_type=jnp.float32))
        u = (jnp.dot(xa, v1b[:D2], preferred_element_type=jnp.float32)
             + jnp.dot(xb, v1b[D2:], preferred_element_type=jnp.float32))
        h = (0.5 * g * (1.0 + lax.erf(g * 0.7071067811865476))) * u
        y = jnp.dot(h.astype(jnp.bfloat16), w2_ref[0].astype(jnp.bfloat16),
                    preferred_element_type=jnp.float32)
        out_ref[...] = _pack_bf16_pair(y[:, :D2], y[:, D2:])


def _gemm(tmeta, xd, w1, v1, w2):
    grid_spec = pltpu.PrefetchScalarGridSpec(
        num_scalar_prefetch=1,
        grid=(NT,),
        in_specs=[
            pl.BlockSpec((M, D2), lambda i, tm: (jnp.minimum(i, tm[24] - 1), 0)),
            pl.BlockSpec((1, D, H), lambda i, tm: (tm[i], 0, 0)),
            pl.BlockSpec((1, D, H), lambda i, tm: (tm[i], 0, 0)),
            pl.BlockSpec((1, H, D), lambda i, tm: (tm[i], 0, 0)),
        ],
        out_specs=pl.BlockSpec((M, D2),
                               lambda i, tm: (jnp.minimum(i, tm[24] - 1), 0)),
    )
    return pl.pallas_call(
        _gemm_body,
        grid_spec=grid_spec,
        out_shape=jax.ShapeDtypeStruct((NP, D2), jnp.int32),
    )(tmeta, xd, w1, v1, w2)


def _combine_body(yd_hbm, inv_hbm, ew_hbm, b_hbm, out_hbm,
                  inv_v, ew_v, bias_v, ybuf, obuf, semg, semo):
    cid = lax.axis_index("c")
    sid = lax.axis_index("s")
    wid = sid * 2 + cid
    NB = TPW // LANES
    pltpu.sync_copy(inv_hbm.at[pl.ds(wid * CPW, CPW)], inv_v)
    pltpu.sync_copy(ew_hbm.at[pl.ds(wid * CPW, CPW)], ew_v.at[pl.ds(0, CPW)])
    pltpu.sync_copy(b_hbm, bias_v)

    def fire(b):
        idx0 = inv_v[pl.ds(b * 32, LANES)]
        idx1 = inv_v[pl.ds(b * 32 + LANES, LANES)]
        s = (b % 2) * 32
        h0 = pltpu.async_copy(yd_hbm.at[idx0],
                              ybuf.at[pl.ds(s, LANES)], semg)
        h1 = pltpu.async_copy(yd_hbm.at[idx1],
                              ybuf.at[pl.ds(s + LANES, LANES)], semg)
        return h0, h1

    hs = {0: fire(0)}
    oh = [None] * NB
    for b in range(NB):
        if b + 1 < NB:
            hs[b + 1] = fire(b + 1)
        hs[b][0].wait()
        hs[b][1].wait()
        if b >= 2:
            oh[b - 2].wait()
        yrow = (b % 2) * 32
        orow = (b % 2) * LANES

        zidx = jnp.zeros((LANES,), jnp.int32)
        oidx = jnp.ones((LANES,), jnp.int32)

        def tok_body(i, carry, b=b, yrow=yrow, orow=orow):
            wv = ew_v[pl.ds(b * 32 + 2 * i, LANES)]
            w0 = _lane_bcast(wv, zidx)
            w1_ = _lane_bcast(wv, oidx)
            r0 = yrow + 2 * i
            ro = orow + i
            for cc in range(D2 // LANES):
                c0 = cc * LANES
                p0 = ybuf[r0, pl.ds(c0, LANES)]
                p1 = ybuf[r0 + 1, pl.ds(c0, LANES)]
                y0a = plsc.bitcast(jnp.left_shift(p0, 16), jnp.float32)
                y1a = plsc.bitcast(jnp.left_shift(p1, 16), jnp.float32)
                y0b = plsc.bitcast(jnp.bitwise_and(p0, -65536), jnp.float32)
                y1b = plsc.bitcast(jnp.bitwise_and(p1, -65536), jnp.float32)
                ba = bias_v[pl.ds(c0, LANES)]
                bb = bias_v[pl.ds(D2 + c0, LANES)]
                obuf[ro, pl.ds(c0, LANES)] = y0a * w0 + y1a * w1_ + ba
                obuf[ro, pl.ds(D2 + c0, LANES)] = y0b * w0 + y1b * w1_ + bb
            return carry

        lax.fori_loop(0, LANES, tok_body, jnp.int32(0))
        oh[b] = pltpu.async_copy(
            obuf.at[pl.ds(orow, LANES)],
            out_hbm.at[pl.ds(wid * TPW + b * LANES, LANES)], semo)
    oh[NB - 2].wait()
    oh[NB - 1].wait()


def kernel(x, router_w, w1, v1, w2, bias):
    t = x.reshape(T, D)
    rwp = jnp.zeros((D, _EPAD), jnp.float32).at[:, :E].set(router_w)
    ew_pad, ei_pad, tb = _router(t, rwp)
    ew = ew_pad.reshape(A)
    ei = ei_pad.reshape(A)
    dispatch, combine = _sc_kernels()
    xd, inv, tmeta = dispatch(ei, tb)
    yd = _gemm(tmeta, xd, w1, v1, w2)
    out = combine(yd, inv, ew, bias)
    return out.reshape(1, T, D)


# speedup vs baseline: 1.2648x; 1.0502x over previous
import functools

import jax
import jax.numpy as jnp
from jax import lax
from jax.experimental import pallas as pl
from jax.experimental.pallas import tpu as pltpu
from jax.experimental.pallas import tpu_sc as plsc

T = 2048
D = 1024
E = 8
H = 1536
K = 2
A = T * K
M = 256
MSH = 8
NP = A + E * M
NT = NP // M
LANES = 16
NW = 32
CPW = A // NW
TPW = T // NW

_EPAD = 128
D2 = D // 2


def _lane_bcast(v, idx):
    dn = lax.GatherDimensionNumbers(
        offset_dims=(), collapsed_slice_dims=(0,), start_index_map=(0,))
    return lax.gather(v, idx[:, None], dn, slice_sizes=(1,),
                      mode=lax.GatherScatterMode.PROMISE_IN_BOUNDS)


def _pack_bf16_pair(lo, hi):
    lo_b = lax.bitcast_convert_type(
        lo.astype(jnp.bfloat16).astype(jnp.float32), jnp.int32)
    hi_b = lax.bitcast_convert_type(
        hi.astype(jnp.bfloat16).astype(jnp.float32), jnp.int32)
    return jnp.bitwise_and(jnp.right_shift(lo_b, 16), 0xFFFF) | hi_b


def _router_body(x_ref, rw_ref, ew_ref, ei_ref, xb_ref):
    xw = x_ref[...]
    xb_ref[...] = _pack_bf16_pair(xw[:, :D2], xw[:, D2:])
    logits = jnp.dot(xw, rw_ref[...], preferred_element_type=jnp.float32)
    lane = lax.broadcasted_iota(jnp.int32, logits.shape, 1)
    valid = lane < E
    logits = jnp.where(valid, logits, -1e30)
    m = jnp.max(logits, axis=1, keepdims=True)
    ex = jnp.where(valid, jnp.exp(logits - m), 0.0)
    sm = ex / jnp.sum(ex, axis=1, keepdims=True)
    v1 = jnp.max(sm, axis=1, keepdims=True)
    i1 = jnp.min(jnp.where(sm == v1, lane, E), axis=1, keepdims=True)
    smm = jnp.where(lane == i1, -1.0, sm)
    v2 = jnp.max(smm, axis=1, keepdims=True)
    i2 = jnp.min(jnp.where(smm == v2, lane, E), axis=1, keepdims=True)
    ew_ref[...] = jnp.concatenate([v1, v2], axis=1)
    ei_ref[...] = jnp.concatenate([i1, i2], axis=1)


def _router(t, rwp):
    return pl.pallas_call(
        _router_body,
        out_shape=(
            jax.ShapeDtypeStruct((T, K), jnp.float32),
            jax.ShapeDtypeStruct((T, K), jnp.int32),
            jax.ShapeDtypeStruct((T, D2), jnp.int32),
        ),
    )(t, rwp)


_NCH = CPW // LANES
_NSLOT = 6


def _dispatch_body(ei_hbm, x_hbm, xd_hbm, inv_hbm, tmeta_hbm,
                   ei_v, inv_v, rowbuf, tmeta_v, semg, sems):
    cid = lax.axis_index("c")
    sid = lax.axis_index("s")
    wid = sid * 2 + cid
    pltpu.sync_copy(ei_hbm, ei_v)
    lanes = lax.iota(jnp.int32, LANES)
    zero = jnp.zeros((LANES,), jnp.int32)
    my_chunk0 = wid * _NCH

    gh = []
    for ci in range(_NSLOT):
        tok = (lanes + wid * CPW + ci * LANES) >> 1
        gh.append(pltpu.async_copy(
            x_hbm.at[tok], rowbuf.at[pl.ds(ci * LANES, LANES)], semg))

    def count_body(c, carry):
        cnt, pref = carry
        pref = jnp.where(jnp.broadcast_to(c == my_chunk0, (LANES,)), cnt, pref)
        v = ei_v[pl.ds(c * LANES, LANES)]
        for e in range(E):
            pc = plsc.all_reduce_population_count(v == e)
            cnt = cnt + jnp.where(lanes == e, pc, 0)
        return cnt, pref

    tot, pref = lax.fori_loop(0, A // LANES, count_body, (zero, zero))

    padded = ((tot + (M - 1)) >> MSH) << MSH
    incl = plsc.cumsum(padded)
    base = (incl - padded) + pref

    sh = [None] * _NCH
    for ci in range(_NCH):
        off = wid * CPW + ci * LANES
        v = ei_v[pl.ds(off, LANES)]
        pos = zero
        for e in range(E):
            msk = v == e
            ones = jnp.where(msk, 1, 0)
            csum = plsc.cumsum(ones)
            be = jnp.sum(jnp.where(lanes == e, base, 0))
            pos = jnp.where(msk, be + csum - 1, pos)
            base = base + jnp.where(lanes == e, jnp.sum(ones), 0)
        inv_v[pl.ds(ci * LANES, LANES)] = pos
        gh[ci].wait()
        slot = ci % _NSLOT
        sh[ci] = pltpu.async_copy(
            rowbuf.at[pl.ds(slot * LANES, LANES)], xd_hbm.at[pos], sems)
        nxt = ci - 1 + _NSLOT
        if ci >= 1 and nxt < _NCH and len(gh) == nxt:
            sh[ci - 1].wait()
            sh[ci - 1] = None
            tok = (lanes + wid * CPW + nxt * LANES) >> 1
            gh.append(pltpu.async_copy(
                x_hbm.at[tok],
                rowbuf.at[pl.ds(((ci - 1) % _NSLOT) * LANES, LANES)], semg))
    for h in sh:
        if h is not None:
            h.wait()
    pltpu.sync_copy(inv_v, inv_hbm.at[pl.ds(wid * CPW, CPW)])

    @pl.when(jnp.logical_and(cid == 0, sid == 0))
    def _():
        j0 = lanes * M
        j1 = (lanes + LANES) * M
        te0 = zero
        te1 = zero
        for e in range(E):
            se = jnp.sum(jnp.where(lanes == e, incl, 0))
            te0 = te0 + jnp.where(j0 >= se, 1, 0)
            te1 = te1 + jnp.where(j1 >= se, 1, 0)
        te0 = jnp.minimum(te0, E - 1)
        te1 = jnp.minimum(te1, E - 1)
        used = jnp.sum(jnp.where(lanes == E - 1, incl, 0)) >> MSH
        te1 = jnp.where(lanes + LANES < NT, te1, used)
        tmeta_v[pl.ds(0, LANES)] = te0
        tmeta_v[pl.ds(LANES, LANES)] = te1
        pltpu.sync_copy(tmeta_v, tmeta_hbm)


@functools.cache
def _sc_kernels():
    mesh = plsc.VectorSubcoreMesh(core_axis_name="c", subcore_axis_name="s")
    dispatch = pl.kernel(
        _dispatch_body,
        out_type=(
            jax.ShapeDtypeStruct((NP, D2), jnp.int32),
            jax.ShapeDtypeStruct((A,), jnp.int32),
            jax.ShapeDtypeStruct((2 * LANES,), jnp.int32),
        ),
        mesh=mesh,
        compiler_params=pltpu.CompilerParams(needs_layout_passes=False),
        scratch_types=[
            pltpu.VMEM((A,), jnp.int32),
            pltpu.VMEM((CPW,), jnp.int32),
            pltpu.VMEM((_NSLOT * LANES, D2), jnp.int32),
            pltpu.VMEM((2 * LANES,), jnp.int32),
            pltpu.SemaphoreType.DMA,
            pltpu.SemaphoreType.DMA,
        ],
    )
    combine = pl.kernel(
        _combine_body,
        out_type=jax.ShapeDtypeStruct((T, D), jnp.float32),
        mesh=mesh,
        compiler_params=pltpu.CompilerParams(needs_layout_passes=False),
        scratch_types=[
            pltpu.VMEM((CPW,), jnp.int32),
            pltpu.VMEM((CPW + LANES,), jnp.float32),
            pltpu.VMEM((D,), jnp.float32),
            pltpu.VMEM((4 * LANES, D2), jnp.int32),
            pltpu.VMEM((2 * LANES, D), jnp.float32),
            pltpu.SemaphoreType.DMA,
            pltpu.SemaphoreType.DMA,
        ],
    )
    return dispatch, combine


def _gemm_body(tmeta_ref, xd_ref, w1_ref, v1_ref, w2_ref, out_ref):
    i = pl.program_id(0)
    used = tmeta_ref[24]

    @pl.when(i < used)
    def _():
        xi = xd_ref[...]
        xa = lax.bitcast_convert_type(
            jnp.left_shift(xi, 16), jnp.float32).astype(jnp.bfloat16)
        xb = lax.bitcast_convert_type(
            jnp.bitwise_and(xi, -65536), jnp.float32).astype(jnp.bfloat16)
        w1b = w1_ref[0].astype(jnp.bfloat16)
        v1b = v1_ref[0].astype(jnp.bfloat16)
        g = (jnp.dot(xa, w1b[:D2], preferred_element_type=jnp.float32)
             + jnp.dot(xb, w1b[D2:], preferred_element_type=jnp.float32))
        u = (jnp.dot(xa, v1b[:D2], preferred_element_type=jnp.float32)
             + jnp.dot(xb, v1b[D2:], preferred_element_type=jnp.float32))
        h = (0.5 * g * (1.0 + lax.erf(g * 0.7071067811865476))) * u
        y = jnp.dot(h.astype(jnp.bfloat16), w2_ref[0].astype(jnp.bfloat16),
                    preferred_element_type=jnp.float32)
        out_ref[...] = _pack_bf16_pair(y[:, :D2], y[:, D2:])


def _gemm(tmeta, xd, w1, v1, w2):
    grid_spec = pltpu.PrefetchScalarGridSpec(
        num_scalar_prefetch=1,
        grid=(NT,),
        in_specs=[
            pl.BlockSpec((M, D2), lambda i, tm: (jnp.minimum(i, tm[24] - 1), 0)),
            pl.BlockSpec((1, D, H), lambda i, tm: (tm[i], 0, 0)),
            pl.BlockSpec((1, D, H), lambda i, tm: (tm[i], 0, 0)),
            pl.BlockSpec((1, H, D), lambda i, tm: (tm[i], 0, 0)),
        ],
        out_specs=pl.BlockSpec((M, D2),
                               lambda i, tm: (jnp.minimum(i, tm[24] - 1), 0)),
    )
    return pl.pallas_call(
        _gemm_body,
        grid_spec=grid_spec,
        out_shape=jax.ShapeDtypeStruct((NP, D2), jnp.int32),
    )(tmeta, xd, w1, v1, w2)


def _combine_body(yd_hbm, inv_hbm, ew_hbm, b_hbm, out_hbm,
                  inv_v, ew_v, bias_v, ybuf, obuf, semg, semo):
    cid = lax.axis_index("c")
    sid = lax.axis_index("s")
    wid = sid * 2 + cid
    NB = TPW // LANES
    pltpu.sync_copy(inv_hbm.at[pl.ds(wid * CPW, CPW)], inv_v)
    pltpu.sync_copy(ew_hbm.at[pl.ds(wid * CPW, CPW)], ew_v.at[pl.ds(0, CPW)])
    pltpu.sync_copy(b_hbm, bias_v)

    def fire(b):
        idx0 = inv_v[pl.ds(b * 32, LANES)]
        idx1 = inv_v[pl.ds(b * 32 + LANES, LANES)]
        s = (b % 2) * 32
        h0 = pltpu.async_copy(yd_hbm.at[idx0],
                              ybuf.at[pl.ds(s, LANES)], semg)
        h1 = pltpu.async_copy(yd_hbm.at[idx1],
                              ybuf.at[pl.ds(s + LANES, LANES)], semg)
        return h0, h1

    hs = {0: fire(0)}
    oh = [None] * NB
    for b in range(NB):
        if b + 1 < NB:
            hs[b + 1] = fire(b + 1)
        hs[b][0].wait()
        hs[b][1].wait()
        if b >= 2:
            oh[b - 2].wait()
        yrow = (b % 2) * 32
        orow = (b % 2) * LANES

        zidx = jnp.zeros((LANES,), jnp.int32)
        oidx = jnp.ones((LANES,), jnp.int32)

        def tok_body(i, b=b, yrow=yrow, orow=orow):
            wv = ew_v[pl.ds(b * 32 + 2 * i, LANES)]
            w0 = _lane_bcast(wv, zidx)
            w1_ = _lane_bcast(wv, oidx)
            r0 = yrow + 2 * i
            ro = orow + i
            for cc in range(D2 // LANES):
                c0 = cc * LANES
                p0 = ybuf[r0, pl.ds(c0, LANES)]
                p1 = ybuf[r0 + 1, pl.ds(c0, LANES)]
                y0a = plsc.bitcast(jnp.left_shift(p0, 16), jnp.float32)
                y1a = plsc.bitcast(jnp.left_shift(p1, 16), jnp.float32)
                y0b = plsc.bitcast(jnp.bitwise_and(p0, -65536), jnp.float32)
                y1b = plsc.bitcast(jnp.bitwise_and(p1, -65536), jnp.float32)
                ba = bias_v[pl.ds(c0, LANES)]
                bb = bias_v[pl.ds(D2 + c0, LANES)]
                obuf[ro, pl.ds(c0, LANES)] = y0a * w0 + y1a * w1_ + ba
                obuf[ro, pl.ds(D2 + c0, LANES)] = y0b * w0 + y1b * w1_ + bb

        plsc.parallel_loop(0, LANES, unroll=2)(tok_body)
        oh[b] = pltpu.async_copy(
            obuf.at[pl.ds(orow, LANES)],
            out_hbm.at[pl.ds(wid * TPW + b * LANES, LANES)], semo)
    oh[NB - 2].wait()
    oh[NB - 1].wait()


def kernel(x, router_w, w1, v1, w2, bias):
    t = x.reshape(T, D)
    rwp = jnp.zeros((D, _EPAD), jnp.float32).at[:, :E].set(router_w)
    ew_pad, ei_pad, tb = _router(t, rwp)
    ew = ew_pad.reshape(A)
    ei = ei_pad.reshape(A)
    dispatch, combine = _sc_kernels()
    xd, inv, tmeta = dispatch(ei, tb)
    yd = _gemm(tmeta, xd, w1, v1, w2)
    out = combine(yd, inv, ew, bias)
    return out.reshape(1, T, D)


# speedup vs baseline: 1.2675x; 1.0021x over previous
import functools

import jax
import jax.numpy as jnp
from jax import lax
from jax.experimental import pallas as pl
from jax.experimental.pallas import tpu as pltpu
from jax.experimental.pallas import tpu_sc as plsc

T = 2048
D = 1024
E = 8
H = 1536
K = 2
A = T * K
M = 256
MSH = 8
NP = A + E * M
NT = NP // M
LANES = 16
NW = 32
CPW = A // NW
TPW = T // NW

_EPAD = 128
D2 = D // 2


def _lane_bcast(v, idx):
    dn = lax.GatherDimensionNumbers(
        offset_dims=(), collapsed_slice_dims=(0,), start_index_map=(0,))
    return lax.gather(v, idx[:, None], dn, slice_sizes=(1,),
                      mode=lax.GatherScatterMode.PROMISE_IN_BOUNDS)


def _pack_bf16_pair(lo, hi):
    lo_b = lax.bitcast_convert_type(
        lo.astype(jnp.bfloat16).astype(jnp.float32), jnp.int32)
    hi_b = lax.bitcast_convert_type(
        hi.astype(jnp.bfloat16).astype(jnp.float32), jnp.int32)
    return jnp.bitwise_and(jnp.right_shift(lo_b, 16), 0xFFFF) | hi_b


def _router_body(x_ref, rw_ref, ew_ref, ei_ref, xb_ref):
    xw = x_ref[...]
    xb_ref[...] = _pack_bf16_pair(xw[:, :D2], xw[:, D2:])
    logits = jnp.dot(xw, rw_ref[...], preferred_element_type=jnp.float32)
    lane = lax.broadcasted_iota(jnp.int32, logits.shape, 1)
    valid = lane < E
    logits = jnp.where(valid, logits, -1e30)
    m = jnp.max(logits, axis=1, keepdims=True)
    ex = jnp.where(valid, jnp.exp(logits - m), 0.0)
    sm = ex / jnp.sum(ex, axis=1, keepdims=True)
    v1 = jnp.max(sm, axis=1, keepdims=True)
    i1 = jnp.min(jnp.where(sm == v1, lane, E), axis=1, keepdims=True)
    smm = jnp.where(lane == i1, -1.0, sm)
    v2 = jnp.max(smm, axis=1, keepdims=True)
    i2 = jnp.min(jnp.where(smm == v2, lane, E), axis=1, keepdims=True)
    ew_ref[...] = jnp.concatenate([v1, v2], axis=1)
    ei_ref[...] = jnp.concatenate([i1, i2], axis=1)


def _router(t, rwp):
    return pl.pallas_call(
        _router_body,
        out_shape=(
            jax.ShapeDtypeStruct((T, K), jnp.float32),
            jax.ShapeDtypeStruct((T, K), jnp.int32),
            jax.ShapeDtypeStruct((T, D2), jnp.int32),
        ),
    )(t, rwp)


_NCH = CPW // LANES
_NSLOT = 6


def _dispatch_body(ei_hbm, x_hbm, xd_hbm, inv_hbm, tmeta_hbm,
                   ei_v, inv_v, rowbuf, tmeta_v, semg, sems):
    cid = lax.axis_index("c")
    sid = lax.axis_index("s")
    wid = sid * 2 + cid
    pltpu.sync_copy(ei_hbm, ei_v)
    lanes = lax.iota(jnp.int32, LANES)
    zero = jnp.zeros((LANES,), jnp.int32)
    my_chunk0 = wid * _NCH

    gh = []
    for ci in range(_NSLOT):
        tok = (lanes + wid * CPW + ci * LANES) >> 1
        gh.append(pltpu.async_copy(
            x_hbm.at[tok], rowbuf.at[pl.ds(ci * LANES, LANES)], semg))

    def count_body(c, carry):
        cnt, pref = carry
        pref = jnp.where(jnp.broadcast_to(c == my_chunk0, (LANES,)), cnt, pref)
        v = ei_v[pl.ds(c * LANES, LANES)]
        for e in range(E):
            pc = plsc.all_reduce_population_count(v == e)
            cnt = cnt + jnp.where(lanes == e, pc, 0)
        return cnt, pref

    tot, pref = plsc.parallel_loop(0, A // LANES, unroll=4,
                                   carry=(zero, zero))(count_body)

    padded = ((tot + (M - 1)) >> MSH) << MSH
    incl = plsc.cumsum(padded)
    base = (incl - padded) + pref

    sh = [None] * _NCH
    for ci in range(_NCH):
        off = wid * CPW + ci * LANES
        v = ei_v[pl.ds(off, LANES)]
        pos = zero
        for e in range(E):
            msk = v == e
            ones = jnp.where(msk, 1, 0)
            csum = plsc.cumsum(ones)
            be = jnp.sum(jnp.where(lanes == e, base, 0))
            pos = jnp.where(msk, be + csum - 1, pos)
            base = base + jnp.where(lanes == e, jnp.sum(ones), 0)
        inv_v[pl.ds(ci * LANES, LANES)] = pos
        gh[ci].wait()
        slot = ci % _NSLOT
        sh[ci] = pltpu.async_copy(
            rowbuf.at[pl.ds(slot * LANES, LANES)], xd_hbm.at[pos], sems)
        nxt = ci - 1 + _NSLOT
        if ci >= 1 and nxt < _NCH and len(gh) == nxt:
            sh[ci - 1].wait()
            sh[ci - 1] = None
            tok = (lanes + wid * CPW + nxt * LANES) >> 1
            gh.append(pltpu.async_copy(
                x_hbm.at[tok],
                rowbuf.at[pl.ds(((ci - 1) % _NSLOT) * LANES, LANES)], semg))
    for h in sh:
        if h is not None:
            h.wait()
    pltpu.sync_copy(inv_v, inv_hbm.at[pl.ds(wid * CPW, CPW)])

    @pl.when(jnp.logical_and(cid == 0, sid == 0))
    def _():
        j0 = lanes * M
        j1 = (lanes + LANES) * M
        te0 = zero
        te1 = zero
        for e in range(E):
            se = jnp.sum(jnp.where(lanes == e, incl, 0))
            te0 = te0 + jnp.where(j0 >= se, 1, 0)
            te1 = te1 + jnp.where(j1 >= se, 1, 0)
        te0 = jnp.minimum(te0, E - 1)
        te1 = jnp.minimum(te1, E - 1)
        used = jnp.sum(jnp.where(lanes == E - 1, incl, 0)) >> MSH
        te1 = jnp.where(lanes + LANES < NT, te1, used)
        tmeta_v[pl.ds(0, LANES)] = te0
        tmeta_v[pl.ds(LANES, LANES)] = te1
        pltpu.sync_copy(tmeta_v, tmeta_hbm)


@functools.cache
def _sc_kernels():
    mesh = plsc.VectorSubcoreMesh(core_axis_name="c", subcore_axis_name="s")
    dispatch = pl.kernel(
        _dispatch_body,
        out_type=(
            jax.ShapeDtypeStruct((NP, D2), jnp.int32),
            jax.ShapeDtypeStruct((A,), jnp.int32),
            jax.ShapeDtypeStruct((2 * LANES,), jnp.int32),
        ),
        mesh=mesh,
        compiler_params=pltpu.CompilerParams(needs_layout_passes=False),
        scratch_types=[
            pltpu.VMEM((A,), jnp.int32),
            pltpu.VMEM((CPW,), jnp.int32),
            pltpu.VMEM((_NSLOT * LANES, D2), jnp.int32),
            pltpu.VMEM((2 * LANES,), jnp.int32),
            pltpu.SemaphoreType.DMA,
            pltpu.SemaphoreType.DMA,
        ],
    )
    combine = pl.kernel(
        _combine_body,
        out_type=jax.ShapeDtypeStruct((T, D), jnp.float32),
        mesh=mesh,
        compiler_params=pltpu.CompilerParams(needs_layout_passes=False),
        scratch_types=[
            pltpu.VMEM((CPW,), jnp.int32),
            pltpu.VMEM((CPW + LANES,), jnp.float32),
            pltpu.VMEM((D,), jnp.float32),
            pltpu.VMEM((4 * LANES, D2), jnp.int32),
            pltpu.VMEM((2 * LANES, D), jnp.float32),
            pltpu.SemaphoreType.DMA,
            pltpu.SemaphoreType.DMA,
        ],
    )
    return dispatch, combine


def _gemm_body(tmeta_ref, xd_ref, w1_ref, v1_ref, w2_ref, out_ref):
    i = pl.program_id(0)
    used = tmeta_ref[24]

    @pl.when(i < used)
    def _():
        xi = xd_ref[...]
        xa = lax.bitcast_convert_type(
            jnp.left_shift(xi, 16), jnp.float32).astype(jnp.bfloat16)
        xb = lax.bitcast_convert_type(
            jnp.bitwise_and(xi, -65536), jnp.float32).astype(jnp.bfloat16)
        w1b = w1_ref[0].astype(jnp.bfloat16)
        v1b = v1_ref[0].astype(jnp.bfloat16)
        g = (jnp.dot(xa, w1b[:D2], preferred_element_type=jnp.float32)
             + jnp.dot(xb, w1b[D2:], preferred_element_type=jnp.float32))
        u = (jnp.dot(xa, v1b[:D2], preferred_element_type=jnp.float32)
             + jnp.dot(xb, v1b[D2:], preferred_element_type=jnp.float32))
        h = (0.5 * g * (1.0 + lax.erf(g * 0.7071067811865476))) * u
        y = jnp.dot(h.astype(jnp.bfloat16), w2_ref[0].astype(jnp.bfloat16),
                    preferred_element_type=jnp.float32)
        out_ref[...] = _pack_bf16_pair(y[:, :D2], y[:, D2:])


def _gemm(tmeta, xd, w1, v1, w2):
    grid_spec = pltpu.PrefetchScalarGridSpec(
        num_scalar_prefetch=1,
        grid=(NT,),
        in_specs=[
            pl.BlockSpec((M, D2), lambda i, tm: (jnp.minimum(i, tm[24] - 1), 0)),
            pl.BlockSpec((1, D, H), lambda i, tm: (tm[i], 0, 0)),
            pl.BlockSpec((1, D, H), lambda i, tm: (tm[i], 0, 0)),
            pl.BlockSpec((1, H, D), lambda i, tm: (tm[i], 0, 0)),
        ],
        out_specs=pl.BlockSpec((M, D2),
                               lambda i, tm: (jnp.minimum(i, tm[24] - 1), 0)),
    )
    return pl.pallas_call(
        _gemm_body,
        grid_spec=grid_spec,
        out_shape=jax.ShapeDtypeStruct((NP, D2), jnp.int32),
    )(tmeta, xd, w1, v1, w2)


def _combine_body(yd_hbm, inv_hbm, ew_hbm, b_hbm, out_hbm,
                  inv_v, ew_v, bias_v, ybuf, obuf, semg, semo):
    cid = lax.axis_index("c")
    sid = lax.axis_index("s")
    wid = sid * 2 + cid
    NB = TPW // LANES
    pltpu.sync_copy(inv_hbm.at[pl.ds(wid * CPW, CPW)], inv_v)
    pltpu.sync_copy(ew_hbm.at[pl.ds(wid * CPW, CPW)], ew_v.at[pl.ds(0, CPW)])
    pltpu.sync_copy(b_hbm, bias_v)

    def fire(b):
        idx0 = inv_v[pl.ds(b * 32, LANES)]
        idx1 = inv_v[pl.ds(b * 32 + LANES, LANES)]
        s = (b % 2) * 32
        h0 = pltpu.async_copy(yd_hbm.at[idx0],
                              ybuf.at[pl.ds(s, LANES)], semg)
        h1 = pltpu.async_copy(yd_hbm.at[idx1],
                              ybuf.at[pl.ds(s + LANES, LANES)], semg)
        return h0, h1

    hs = {0: fire(0)}
    oh = [None] * NB
    for b in range(NB):
        if b + 1 < NB:
            hs[b + 1] = fire(b + 1)
        hs[b][0].wait()
        hs[b][1].wait()
        if b >= 2:
            oh[b - 2].wait()
        yrow = (b % 2) * 32
        orow = (b % 2) * LANES

        zidx = jnp.zeros((LANES,), jnp.int32)
        oidx = jnp.ones((LANES,), jnp.int32)

        def tok_body(i, b=b, yrow=yrow, orow=orow):
            wv = ew_v[pl.ds(b * 32 + 2 * i, LANES)]
            w0 = _lane_bcast(wv, zidx)
            w1_ = _lane_bcast(wv, oidx)
            r0 = yrow + 2 * i
            ro = orow + i
            for cc in range(D2 // LANES):
                c0 = cc * LANES
                p0 = ybuf[r0, pl.ds(c0, LANES)]
                p1 = ybuf[r0 + 1, pl.ds(c0, LANES)]
                y0a = plsc.bitcast(jnp.left_shift(p0, 16), jnp.float32)
                y1a = plsc.bitcast(jnp.left_shift(p1, 16), jnp.float32)
                y0b = plsc.bitcast(jnp.bitwise_and(p0, -65536), jnp.float32)
                y1b = plsc.bitcast(jnp.bitwise_and(p1, -65536), jnp.float32)
                ba = bias_v[pl.ds(c0, LANES)]
                bb = bias_v[pl.ds(D2 + c0, LANES)]
                obuf[ro, pl.ds(c0, LANES)] = y0a * w0 + y1a * w1_ + ba
                obuf[ro, pl.ds(D2 + c0, LANES)] = y0b * w0 + y1b * w1_ + bb

        plsc.parallel_loop(0, LANES, unroll=2)(tok_body)
        oh[b] = pltpu.async_copy(
            obuf.at[pl.ds(orow, LANES)],
            out_hbm.at[pl.ds(wid * TPW + b * LANES, LANES)], semo)
    oh[NB - 2].wait()
    oh[NB - 1].wait()


def kernel(x, router_w, w1, v1, w2, bias):
    t = x.reshape(T, D)
    rwp = jnp.zeros((D, _EPAD), jnp.float32).at[:, :E].set(router_w)
    ew_pad, ei_pad, tb = _router(t, rwp)
    ew = ew_pad.reshape(A)
    ei = ei_pad.reshape(A)
    dispatch, combine = _sc_kernels()
    xd, inv, tmeta = dispatch(ei, tb)
    yd = _gemm(tmeta, xd, w1, v1, w2)
    out = combine(yd, inv, ew, bias)
    return out.reshape(1, T, D)
